# baseline (speedup 1.0000x reference)
import functools

import jax
import jax.numpy as jnp
from jax.experimental import pallas as pl
from jax.experimental.pallas import tpu as pltpu

_R = 64
_K = 128
_C = _R + _R * _K
_B = 4096


def _body(x_ref, t_ref, la_ref, ela_ref, clone_ref, loss_ref):
    x = x_ref[...]
    t = t_ref[...]
    la = la_ref[...]
    ela = ela_ref[...]
    BC = x.shape[1]

    m = jnp.max(x, axis=0, keepdims=True)
    xs = x - m
    E = jnp.exp(xs)
    Ea = E * ela

    dot_xa_t = jnp.sum((xs + la) * t)

    sp = jnp.sum(E[:_R], axis=0, keepdims=True)
    spa = jnp.sum(Ea[:_R], axis=0, keepdims=True)
    tp = jnp.sum(t[:_R], axis=0, keepdims=True)
    epo_par = E[:_R] / sp
    lse_dot = jnp.sum(jnp.log(spa) * tp)

    E3 = E[_R:].reshape(_R, _K, BC)
    Ea3 = Ea[_R:].reshape(_R, _K, BC)
    t3 = t[_R:].reshape(_R, _K, BC)
    s3 = jnp.sum(E3, axis=1, keepdims=True)
    sa3 = jnp.sum(Ea3, axis=1, keepdims=True)
    tg3 = jnp.sum(t3, axis=1, keepdims=True)
    clone3 = E3 * (epo_par.reshape(_R, 1, BC) / s3)
    lse_dot = lse_dot + jnp.sum(jnp.log(sa3) * tg3)

    clone_ref[:_R, :] = epo_par
    clone_ref[_R:, :] = clone3.reshape(_R * _K, BC)
    loss_part = dot_xa_t - lse_dot

    @pl.when(pl.program_id(0) == 0)
    def _init():
        loss_ref[0, 0] = 0.0

    loss_ref[0, 0] += loss_part


@functools.partial(jax.jit, static_argnames=("interpret",))
def kernel(pred, target, logit_adjustment, interpret=False):
    BC = 128
    xT = pred.T
    tT = target.T
    laT = jnp.broadcast_to(logit_adjustment.reshape(_C, 1), (_C, BC))
    elaT = jnp.exp(laT)
    cloneT, acc = pl.pallas_call(
        _body,
        grid=(_B // BC,),
        in_specs=[
            pl.BlockSpec((_C, BC), lambda b: (0, b)),
            pl.BlockSpec((_C, BC), lambda b: (0, b)),
            pl.BlockSpec((_C, BC), lambda b: (0, 0)),
            pl.BlockSpec((_C, BC), lambda b: (0, 0)),
        ],
        out_specs=[
            pl.BlockSpec((_C, BC), lambda b: (0, b)),
            pl.BlockSpec(memory_space=pltpu.SMEM, block_shape=(1, 1),
                         index_map=lambda b: (0, 0)),
        ],
        out_shape=[
            jax.ShapeDtypeStruct((_C, _B), jnp.float32),
            jax.ShapeDtypeStruct((1, 1), jnp.float32),
        ],
        interpret=interpret,
    )(xT, tT, laT, elaT)
    loss = -acc[0, 0] / _B
    return (loss, cloneT.T)


# speedup vs baseline: 1.1592x; 1.1592x over previous
import functools

import jax
import jax.numpy as jnp
from jax.experimental import pallas as pl
from jax.experimental.pallas import tpu as pltpu

_R = 64
_K = 128
_C = _R + _R * _K
_B = 4096


def _body(x_ref, t_ref, la_ref, ela_ref, clone_ref, loss_ref):
    x = x_ref[...]
    t = t_ref[...]
    la = la_ref[...]
    ela = ela_ref[...]
    BC = x.shape[1]

    m = jnp.max(x, axis=0, keepdims=True)
    xs = x - m
    E = jnp.exp(xs)
    Ea = E * ela

    dot_xa_t = jnp.sum((xs + la) * t)

    sp = jnp.sum(E[:_R], axis=0, keepdims=True)
    spa = jnp.sum(Ea[:_R], axis=0, keepdims=True)
    tp = jnp.sum(t[:_R], axis=0, keepdims=True)
    epo_par = E[:_R] / sp
    lse_dot = jnp.sum(jnp.log(spa) * tp)

    E3 = E[_R:].reshape(_R, _K, BC)
    Ea3 = Ea[_R:].reshape(_R, _K, BC)
    t3 = t[_R:].reshape(_R, _K, BC)
    s3 = jnp.sum(E3, axis=1, keepdims=True)
    sa3 = jnp.sum(Ea3, axis=1, keepdims=True)
    tg3 = jnp.sum(t3, axis=1, keepdims=True)
    clone3 = E3 * (epo_par.reshape(_R, 1, BC) / s3)
    lse_dot = lse_dot + jnp.sum(jnp.log(sa3) * tg3)

    clone_ref[:_R, :] = epo_par
    clone_ref[_R:, :] = clone3.reshape(_R * _K, BC)
    loss_part = dot_xa_t - lse_dot

    @pl.when(pl.program_id(0) == 0)
    def _init():
        loss_ref[0, 0] = 0.0

    loss_ref[0, 0] += loss_part


@functools.partial(jax.jit, static_argnames=("interpret",))
def kernel(pred, target, logit_adjustment, interpret=False):
    BC = 128
    xT = pred.T
    tT = target.T
    laT = logit_adjustment.reshape(_C, 1)
    elaT = jnp.exp(laT)
    cloneT, acc = pl.pallas_call(
        _body,
        grid=(_B // BC,),
        in_specs=[
            pl.BlockSpec((_C, BC), lambda b: (0, b)),
            pl.BlockSpec((_C, BC), lambda b: (0, b)),
            pl.BlockSpec((_C, 1), lambda b: (0, 0)),
            pl.BlockSpec((_C, 1), lambda b: (0, 0)),
        ],
        out_specs=[
            pl.BlockSpec((_C, BC), lambda b: (0, b)),
            pl.BlockSpec(memory_space=pltpu.SMEM, block_shape=(1, 1),
                         index_map=lambda b: (0, 0)),
        ],
        out_shape=[
            jax.ShapeDtypeStruct((_C, _B), jnp.float32),
            jax.ShapeDtypeStruct((1, 1), jnp.float32),
        ],
        interpret=interpret,
    )(xT, tT, laT, elaT)
    loss = -acc[0, 0] / _B
    return (loss, cloneT.T)
